# Initial kernel scaffold; baseline (speedup 1.0000x reference)
import jax
import jax.numpy as jnp
from jax.experimental import pallas as pl


def kernel(world_pos, mesh_pos, prev_world_pos, phi, prev_phi, swelling_phi, swelling_phi_rate, swelling_phi_rate_prev, node_type, mat_param, edge_index, params):
    raise NotImplementedError("write your pallas kernel here")


# speedup vs baseline: 4.5390x; 4.5390x over previous
import functools

import jax
import jax.numpy as jnp
from jax import lax
from jax.experimental import pallas as pl
from jax.experimental.pallas import tpu as pltpu
from jax.experimental.pallas import tpu_sc as plsc

N = 10000
NPAD = 10240
E = 320000
H = 128
NW = 32
EPW = E // NW
KC = 80
NCHUNK = EPW // KC
NBLK = 2048
EBLK = 2560

_MESH = dict(core_axis_name="c", subcore_axis_name="s")


def _gather2(table, idxa, idxb, width):
    def body(table_ref, ia_ref, ib_ref, outa_ref, outb_ref,
             ia_v, ib_v, buf_a, buf_b, sem_a, sem_b):
        cid = lax.axis_index("c")
        sid = lax.axis_index("s")
        wid = sid * 2 + cid
        pltpu.sync_copy(ia_ref.at[wid], ia_v)
        pltpu.sync_copy(ib_ref.at[wid], ib_v)

        def step(j, carry):
            base = wid * EPW + j * KC
            ca = pltpu.async_copy(table_ref.at[ia_v.at[j]], buf_a, sem_a)
            cb = pltpu.async_copy(table_ref.at[ib_v.at[j]], buf_b, sem_b)
            ca.wait()
            cb.wait()
            pltpu.sync_copy(buf_a, outa_ref.at[pl.ds(base, KC)])
            pltpu.sync_copy(buf_b, outb_ref.at[pl.ds(base, KC)])
            return carry

        lax.fori_loop(0, NCHUNK, step, 0)

    out = jax.ShapeDtypeStruct((E, width), jnp.float32)
    return pl.kernel(
        body,
        out_type=(out, out),
        mesh=plsc.VectorSubcoreMesh(**_MESH),
        scratch_types=(
            pltpu.VMEM((NCHUNK, KC), jnp.int32),
            pltpu.VMEM((NCHUNK, KC), jnp.int32),
            pltpu.VMEM((KC, width), jnp.float32),
            pltpu.VMEM((KC, width), jnp.float32),
            pltpu.SemaphoreType.DMA,
            pltpu.SemaphoreType.DMA,
        ),
        name=f"sc_gather2_{width}",
    )(table, idxa, idxb)


def _scatter_add(msg, idxd, zeros):
    def body(msg_ref, idx_ref, zeros_ref, out_ref, idx_v, mbuf, aggr_sh):
        cid = lax.axis_index("c")
        sid = lax.axis_index("s")
        wid = sid * 2 + cid

        @pl.when(sid == 0)
        def _():
            pltpu.sync_copy(zeros_ref, aggr_sh)

        plsc.subcore_barrier()
        pltpu.sync_copy(idx_ref.at[wid], idx_v)

        def step(j, carry):
            base = wid * EPW + j * KC
            pltpu.sync_copy(msg_ref.at[pl.ds(base, KC)], mbuf)
            pltpu.sync_copy(mbuf, aggr_sh.at[idx_v.at[j]], add=True)
            return carry

        lax.fori_loop(0, NCHUNK, step, 0)
        plsc.subcore_barrier()
        rows = NPAD // 16
        pltpu.sync_copy(aggr_sh.at[pl.ds(sid * rows, rows)],
                        out_ref.at[pl.ds(cid * NPAD + sid * rows, rows)])

    return pl.kernel(
        body,
        out_type=jax.ShapeDtypeStruct((2 * NPAD, H), jnp.float32),
        mesh=plsc.VectorSubcoreMesh(**_MESH),
        scratch_types=(
            pltpu.VMEM((NCHUNK, KC), jnp.int32),
            pltpu.VMEM((KC, H), jnp.float32),
            pltpu.VMEM_SHARED((NPAD, H), jnp.float32),
        ),
        name="sc_scatter_add",
    )(msg, idxd, zeros)


def _ln(y, g, beta):
    mu = jnp.mean(y, axis=-1, keepdims=True)
    var = jnp.mean((y - mu) ** 2, axis=-1, keepdims=True)
    return (y - mu) * lax.rsqrt(var + 1e-5) * g + beta


def _dot(a, b):
    return jnp.dot(a, b, preferred_element_type=jnp.float32)


def _nspec(w=H):
    return pl.BlockSpec((NBLK, w), lambda i: (i, 0))


def _espec(w=H):
    return pl.BlockSpec((EBLK, w), lambda i: (i, 0))


def _wspec(r, c):
    return pl.BlockSpec((r, c), lambda i: (0, 0))


def _encode_nodes(x, w1, b1, w2, b2, g, beta, w1a, w1b):
    def body(x_ref, w1_ref, b1_ref, w2_ref, b2_ref, g_ref, be_ref,
             w1a_ref, w1b_ref, xh_ref, c_ref):
        h = jnp.maximum(_dot(x_ref[...], w1_ref[...]) + b1_ref[...], 0.0)
        y = _dot(h, w2_ref[...]) + b2_ref[...]
        xh = _ln(y, g_ref[...], be_ref[...])
        xh_ref[...] = xh
        c_ref[...] = jnp.concatenate(
            [_dot(xh, w1a_ref[...]), _dot(xh, w1b_ref[...])], axis=1)

    return pl.pallas_call(
        body,
        grid=(NPAD // NBLK,),
        in_specs=[_nspec(16), _wspec(16, H), _wspec(1, H), _wspec(H, H),
                  _wspec(1, H), _wspec(1, H), _wspec(1, H), _wspec(H, H),
                  _wspec(H, H)],
        out_specs=[_nspec(H), _nspec(2 * H)],
        out_shape=[jax.ShapeDtypeStruct((NPAD, H), jnp.float32),
                   jax.ShapeDtypeStruct((NPAD, 2 * H), jnp.float32)],
    )(x, w1, b1, w2, b2, g, beta, w1a, w1b)


def _edge_step0(gd, gs, we1, be1, we2, be2, ge, bee, w1c, b1, w2, b2, g, beta):
    def body(gd_ref, gs_ref, we1_ref, be1_ref, we2_ref, be2_ref, ge_ref,
             bee_ref, w1c_ref, b1_ref, w2_ref, b2_ref, g_ref, be_ref,
             msg_ref, enew_ref):
        gd = gd_ref[...]
        gs = gs_ref[...]
        d = gs[:, 2 * H:2 * H + 16] - gd[:, 2 * H:2 * H + 16]
        dist = jnp.sqrt(d[:, 0:1] ** 2 + d[:, 1:2] ** 2)
        dw = jnp.sqrt(d[:, 2:3] ** 2 + d[:, 3:4] ** 2)
        lane = lax.broadcasted_iota(jnp.int32, d.shape, 1)
        feat = jnp.where(lane == 5, dist, d)
        feat = jnp.where(lane == 6, dw, feat)
        h = jnp.maximum(_dot(feat, we1_ref[...]) + be1_ref[...], 0.0)
        y = _dot(h, we2_ref[...]) + be2_ref[...]
        eh_v = _ln(y, ge_ref[...], bee_ref[...])
        e1 = _dot(eh_v, w1c_ref[...]) + b1_ref[...]
        h1 = jnp.maximum(gd[:, :H] + gs[:, H:2 * H] + e1, 0.0)
        y1 = _dot(h1, w2_ref[...]) + b2_ref[...]
        msg_ref[...] = _ln(y1, g_ref[...], be_ref[...])
        h2 = jnp.maximum(gs[:, :H] + gd[:, H:2 * H] + e1, 0.0)
        y2 = _dot(h2, w2_ref[...]) + b2_ref[...]
        enew_ref[...] = eh_v + _ln(y2, g_ref[...], be_ref[...])

    return pl.pallas_call(
        body,
        grid=(E // EBLK,),
        in_specs=[_espec(3 * H), _espec(3 * H), _wspec(16, H), _wspec(1, H),
                  _wspec(H, H), _wspec(1, H), _wspec(1, H), _wspec(1, H),
                  _wspec(H, H), _wspec(1, H), _wspec(H, H), _wspec(1, H),
                  _wspec(1, H), _wspec(1, H)],
        out_specs=[_espec(H), _espec(H)],
        out_shape=[jax.ShapeDtypeStruct((E, H), jnp.float32),
                   jax.ShapeDtypeStruct((E, H), jnp.float32)],
    )(gd, gs, we1, be1, we2, be2, ge, bee, w1c, b1, w2, b2, g, beta)


def _edge_step(eh, gd, gs, w1c, b1, w2, b2, g, beta):
    def body(eh_ref, gd_ref, gs_ref, w1c_ref, b1_ref, w2_ref, b2_ref,
             g_ref, be_ref, msg_ref, enew_ref):
        eh_v = eh_ref[...]
        e1 = _dot(eh_v, w1c_ref[...]) + b1_ref[...]
        gd = gd_ref[...]
        gs = gs_ref[...]
        h1 = jnp.maximum(gd[:, :H] + gs[:, H:] + e1, 0.0)
        y1 = _dot(h1, w2_ref[...]) + b2_ref[...]
        msg_ref[...] = _ln(y1, g_ref[...], be_ref[...])
        h2 = jnp.maximum(gs[:, :H] + gd[:, H:] + e1, 0.0)
        y2 = _dot(h2, w2_ref[...]) + b2_ref[...]
        enew_ref[...] = eh_v + _ln(y2, g_ref[...], be_ref[...])

    return pl.pallas_call(
        body,
        grid=(E // EBLK,),
        in_specs=[_espec(H), _espec(2 * H), _espec(2 * H), _wspec(H, H),
                  _wspec(1, H), _wspec(H, H), _wspec(1, H), _wspec(1, H),
                  _wspec(1, H)],
        out_specs=[_espec(H), _espec(H)],
        out_shape=[jax.ShapeDtypeStruct((E, H), jnp.float32),
                   jax.ShapeDtypeStruct((E, H), jnp.float32)],
    )(eh, gd, gs, w1c, b1, w2, b2, g, beta)


def _node_step(aggr2, xh, w1na, w1nb, b1, w2, b2, g, beta, w1a, w1b):
    def body(a_ref, xh_ref, w1na_ref, w1nb_ref, b1_ref, w2_ref, b2_ref,
             g_ref, be_ref, w1a_ref, w1b_ref, xn_ref, c_ref):
        aggr = a_ref[0] + a_ref[1]
        xh_v = xh_ref[...]
        pre = _dot(aggr, w1na_ref[...]) + _dot(xh_v, w1nb_ref[...]) + b1_ref[...]
        h = jnp.maximum(pre, 0.0)
        y = _dot(h, w2_ref[...]) + b2_ref[...]
        xn = xh_v + _ln(y, g_ref[...], be_ref[...])
        xn_ref[...] = xn
        c_ref[...] = jnp.concatenate(
            [_dot(xn, w1a_ref[...]), _dot(xn, w1b_ref[...])], axis=1)

    aspec = pl.BlockSpec((2, NBLK, H), lambda i: (0, i, 0))
    return pl.pallas_call(
        body,
        grid=(NPAD // NBLK,),
        in_specs=[aspec, _nspec(H), _wspec(H, H), _wspec(H, H), _wspec(1, H),
                  _wspec(H, H), _wspec(1, H), _wspec(1, H), _wspec(1, H),
                  _wspec(H, H), _wspec(H, H)],
        out_specs=[_nspec(H), _nspec(2 * H)],
        out_shape=[jax.ShapeDtypeStruct((NPAD, H), jnp.float32),
                   jax.ShapeDtypeStruct((NPAD, 2 * H), jnp.float32)],
    )(aggr2, xh, w1na, w1nb, b1, w2, b2, g, beta, w1a, w1b)


def _node_step_last(aggr2, xh, w1na, w1nb, b1, w2, b2, g, beta,
                    wd1a, bd1a, wd2a, wd1b, bd1b, wd2b, bdec):
    def body(a_ref, xh_ref, w1na_ref, w1nb_ref, b1_ref, w2_ref, b2_ref,
             g_ref, be_ref, wd1a_ref, bd1a_ref, wd2a_ref, wd1b_ref, bd1b_ref,
             wd2b_ref, bdec_ref, out_ref):
        aggr = a_ref[0] + a_ref[1]
        xh_v = xh_ref[...]
        pre = _dot(aggr, w1na_ref[...]) + _dot(xh_v, w1nb_ref[...]) + b1_ref[...]
        h = jnp.maximum(pre, 0.0)
        y = _dot(h, w2_ref[...]) + b2_ref[...]
        xn = xh_v + _ln(y, g_ref[...], be_ref[...])
        h1 = jnp.maximum(_dot(xn, wd1a_ref[...]) + bd1a_ref[...], 0.0)
        h2 = jnp.maximum(_dot(xn, wd1b_ref[...]) + bd1b_ref[...], 0.0)
        out_ref[...] = _dot(h1, wd2a_ref[...]) + _dot(h2, wd2b_ref[...]) \
            + bdec_ref[...]

    aspec = pl.BlockSpec((2, NBLK, H), lambda i: (0, i, 0))
    return pl.pallas_call(
        body,
        grid=(NPAD // NBLK,),
        in_specs=[aspec, _nspec(H), _wspec(H, H), _wspec(H, H), _wspec(1, H),
                  _wspec(H, H), _wspec(1, H), _wspec(1, H), _wspec(1, H),
                  _wspec(H, H), _wspec(1, H), _wspec(H, H), _wspec(H, H),
                  _wspec(1, H), _wspec(H, H), _wspec(1, H)],
        out_specs=_nspec(H),
        out_shape=jax.ShapeDtypeStruct((NPAD, H), jnp.float32),
    )(aggr2, xh, w1na, w1nb, b1, w2, b2, g, beta,
      wd1a, bd1a, wd2a, wd1b, bd1b, wd2b, bdec)


def _row(v):
    return v.reshape(1, -1)


def _padn(a):
    return jnp.pad(a, ((0, NPAD - N), (0, 0)))


def kernel(world_pos, mesh_pos, prev_world_pos, phi, prev_phi, swelling_phi,
           swelling_phi_rate, swelling_phi_rate_prev, node_type, mat_param,
           edge_index, params):
    f32 = jnp.float32
    src = edge_index[0].astype(jnp.int32).reshape(NW, NCHUNK, KC)
    dst = edge_index[1].astype(jnp.int32).reshape(NW, NCHUNK, KC)

    x = jnp.concatenate(
        [world_pos - prev_world_pos, phi - prev_phi, swelling_phi,
         swelling_phi_rate, swelling_phi_rate_prev, node_type], axis=-1)
    x = _padn(jnp.pad(x, ((0, 0), (0, 6)))).astype(f32)

    p_tab = _padn(jnp.pad(
        jnp.concatenate([mesh_pos, world_pos, phi], axis=-1),
        ((0, 0), (0, H - 5)))).astype(f32)

    ne = params["node_encoder"]
    ee = params["edge_encoder"]
    proc = params["proc"]

    wn1 = jnp.pad(ne["W1"], ((0, 6), (0, 0)))
    we1 = jnp.pad(ee["W1"][jnp.array([0, 1, 3, 4, 6, 2, 5]), :],
                  ((0, 9), (0, 0)))

    ew = [p["edge_mlp"] for p in proc]
    nw_ = [p["node_mlp"] for p in proc]
    w1a = [w["W1"][:H] for w in ew]
    w1b = [w["W1"][H:2 * H] for w in ew]
    w1c = [w["W1"][2 * H:] for w in ew]

    xh, c = _encode_nodes(x, wn1, _row(ne["b1"]), ne["W2"], _row(ne["b2"]),
                          _row(ne["g"]), _row(ne["beta"]), w1a[0], w1b[0])

    zeros = jnp.zeros((NPAD, H), f32)
    wd = params["world_pos_decoder"]
    pdx = params["phi_decoder"]
    wd2a = jnp.pad(wd["W2"], ((0, 0), (0, H - 2)))
    wd2b = jnp.pad(pdx["W2"], ((0, 0), (2, H - 3)))
    bdec = _row(jnp.pad(jnp.concatenate([wd["b2"], pdx["b2"]]), (0, H - 3)))

    for i in range(3):
        e = ew[i]
        nm = nw_[i]
        if i == 0:
            t0 = jnp.concatenate([c, p_tab], axis=1)
            gd, gs = _gather2(t0, dst, src, 3 * H)
            msg, eh = _edge_step0(
                gd, gs, we1, _row(ee["b1"]), ee["W2"], _row(ee["b2"]),
                _row(ee["g"]), _row(ee["beta"]), w1c[0], _row(e["b1"]),
                e["W2"], _row(e["b2"]), _row(e["g"]), _row(e["beta"]))
        else:
            gd, gs = _gather2(c, dst, src, 2 * H)
            msg, eh = _edge_step(eh, gd, gs, w1c[i], _row(e["b1"]), e["W2"],
                                 _row(e["b2"]), _row(e["g"]), _row(e["beta"]))
        aggr2 = _scatter_add(msg, dst, zeros).reshape(2, NPAD, H)
        if i < 2:
            xh, c = _node_step(aggr2, xh, nm["W1"][:H], nm["W1"][H:],
                               _row(nm["b1"]), nm["W2"], _row(nm["b2"]),
                               _row(nm["g"]), _row(nm["beta"]),
                               w1a[i + 1], w1b[i + 1])
        else:
            out = _node_step_last(aggr2, xh, nm["W1"][:H], nm["W1"][H:],
                                  _row(nm["b1"]), nm["W2"], _row(nm["b2"]),
                                  _row(nm["g"]), _row(nm["beta"]),
                                  wd["W1"], _row(wd["b1"]), wd2a,
                                  pdx["W1"], _row(pdx["b1"]), wd2b, bdec)
    return out[:N, :3]


# speedup vs baseline: 5.7096x; 1.2579x over previous
import functools

import jax
import jax.numpy as jnp
from jax import lax
from jax.experimental import pallas as pl
from jax.experimental.pallas import tpu as pltpu
from jax.experimental.pallas import tpu_sc as plsc

N = 10000
NPAD = 10240
E = 320000
H = 128
NW = 32
EPW = E // NW
KC = 40
NCHUNK = EPW // KC
NBLK = 2048
EBLK = 2560

_MESH = dict(core_axis_name="c", subcore_axis_name="s")


def _gather_combine(table, idxa, idxb, width):
    ngrp = 17 if width > 2 * H else 16

    def body(table_ref, ia_ref, ib_ref, out_ref,
             ia_v, ib_v, g0, bb0, g1, bb1, sg0, sg1, ss0, ss1):
        cid = lax.axis_index("c")
        sid = lax.axis_index("s")
        wid = sid * 2 + cid
        pltpu.sync_copy(ia_ref.at[wid], ia_v)
        pltpu.sync_copy(ib_ref.at[wid], ib_v)
        bufs = ((g0, bb0, sg0, ss0), (g1, bb1, sg1, ss1))

        def issue(j, p):
            g, bb, sg, _ = bufs[p]
            pltpu.async_copy(table_ref.at[ia_v.at[j]], g, sg)
            pltpu.async_copy(table_ref.at[ib_v.at[j]], bb, sg)

        def wait_gather(p):
            g, bb, sg, _ = bufs[p]
            pltpu.make_async_copy(table_ref.at[ia_v.at[0]], g, sg).wait()
            pltpu.make_async_copy(table_ref.at[ib_v.at[0]], bb, sg).wait()

        def wait_store(p):
            g, _, _, ss = bufs[p]
            pltpu.make_async_copy(g, out_ref.at[pl.ds(0, KC)], ss).wait()

        def combine_store(j, p):
            g, bb, _, ss = bufs[p]

            def row(r, carry):
                for gr in range(ngrp):
                    sl = pl.ds(gr * 16, 16)
                    if gr < 8:
                        g[r, sl] = g[r, sl] + bb[r, pl.ds(gr * 16 + H, 16)]
                    elif gr < 16:
                        g[r, sl] = g[r, sl] + bb[r, pl.ds(gr * 16 - H, 16)]
                    else:
                        g[r, sl] = bb[r, sl] - g[r, sl]
                return carry

            lax.fori_loop(0, KC, row, 0)
            pltpu.async_copy(g, out_ref.at[pl.ds(wid * EPW + j * KC, KC)], ss)

        issue(0, 0)

        def step(j2, carry):
            @pl.when(j2 > 0)
            def _():
                wait_store(1)

            issue(2 * j2 + 1, 1)
            wait_gather(0)
            combine_store(2 * j2, 0)

            @pl.when(j2 + 1 < NCHUNK // 2)
            def _():
                wait_store(0)
                issue(2 * j2 + 2, 0)

            wait_gather(1)
            combine_store(2 * j2 + 1, 1)
            return carry

        lax.fori_loop(0, NCHUNK // 2, step, 0)
        wait_store(0)
        wait_store(1)

    fbuf = pltpu.VMEM((KC, width), jnp.float32)
    return pl.kernel(
        body,
        out_type=jax.ShapeDtypeStruct((E, width), jnp.float32),
        mesh=plsc.VectorSubcoreMesh(**_MESH),
        scratch_types=(
            pltpu.VMEM((NCHUNK, KC), jnp.int32),
            pltpu.VMEM((NCHUNK, KC), jnp.int32),
            fbuf, fbuf, fbuf, fbuf,
            pltpu.SemaphoreType.DMA,
            pltpu.SemaphoreType.DMA,
            pltpu.SemaphoreType.DMA,
            pltpu.SemaphoreType.DMA,
        ),
        name=f"sc_gather_combine_{width}",
    )(table, idxa, idxb)


def _scatter_add(msg, idxd, zeros):
    def body(msg_ref, idx_ref, zeros_ref, out_ref,
             idx_v, m0, m1, sl0, sl1, ss0, ss1, aggr_sh):
        cid = lax.axis_index("c")
        sid = lax.axis_index("s")
        wid = sid * 2 + cid

        @pl.when(sid == 0)
        def _():
            pltpu.sync_copy(zeros_ref, aggr_sh)

        plsc.subcore_barrier()
        pltpu.sync_copy(idx_ref.at[wid], idx_v)
        bufs = ((m0, sl0, ss0), (m1, sl1, ss1))

        def load(j, p):
            m, sl, _ = bufs[p]
            pltpu.async_copy(msg_ref.at[pl.ds(wid * EPW + j * KC, KC)], m, sl)

        def wait_load(p):
            m, sl, _ = bufs[p]
            pltpu.make_async_copy(msg_ref.at[pl.ds(0, KC)], m, sl).wait()

        def scat(j, p):
            m, _, ss = bufs[p]
            pltpu.async_copy(m, aggr_sh.at[idx_v.at[j]], ss, add=True)

        def wait_scat(p):
            m, _, ss = bufs[p]
            pltpu.make_async_copy(m, aggr_sh.at[idx_v.at[0]], ss).wait()

        load(0, 0)

        def step(j2, carry):
            load(2 * j2 + 1, 1)
            wait_load(0)
            scat(2 * j2, 0)
            wait_scat(0)

            @pl.when(j2 + 1 < NCHUNK // 2)
            def _():
                load(2 * j2 + 2, 0)

            wait_load(1)
            scat(2 * j2 + 1, 1)
            wait_scat(1)
            return carry

        lax.fori_loop(0, NCHUNK // 2, step, 0)
        plsc.subcore_barrier()
        rows = NPAD // 16
        pltpu.sync_copy(aggr_sh.at[pl.ds(sid * rows, rows)],
                        out_ref.at[pl.ds(cid * NPAD + sid * rows, rows)])

    mbuf = pltpu.VMEM((KC, H), jnp.float32)
    return pl.kernel(
        body,
        out_type=jax.ShapeDtypeStruct((2 * NPAD, H), jnp.float32),
        mesh=plsc.VectorSubcoreMesh(**_MESH),
        scratch_types=(
            pltpu.VMEM((NCHUNK, KC), jnp.int32),
            mbuf, mbuf,
            pltpu.SemaphoreType.DMA,
            pltpu.SemaphoreType.DMA,
            pltpu.SemaphoreType.DMA,
            pltpu.SemaphoreType.DMA,
            pltpu.VMEM_SHARED((NPAD, H), jnp.float32),
        ),
        name="sc_scatter_add",
    )(msg, idxd, zeros)


def _ln(y, g, beta):
    mu = jnp.mean(y, axis=-1, keepdims=True)
    var = jnp.mean((y - mu) ** 2, axis=-1, keepdims=True)
    return (y - mu) * lax.rsqrt(var + 1e-5) * g + beta


def _dot(a, b):
    return jnp.dot(a, b, preferred_element_type=jnp.float32)


def _nspec(w=H):
    return pl.BlockSpec((NBLK, w), lambda i: (i, 0))


def _espec(w=H):
    return pl.BlockSpec((EBLK, w), lambda i: (i, 0))


def _wspec(r, c):
    return pl.BlockSpec((r, c), lambda i: (0, 0))


def _encode_nodes(x, w1, b1, w2, b2, g, beta, w1a, w1b):
    def body(x_ref, w1_ref, b1_ref, w2_ref, b2_ref, g_ref, be_ref,
             w1a_ref, w1b_ref, xh_ref, c_ref):
        h = jnp.maximum(_dot(x_ref[...], w1_ref[...]) + b1_ref[...], 0.0)
        y = _dot(h, w2_ref[...]) + b2_ref[...]
        xh = _ln(y, g_ref[...], be_ref[...])
        xh_ref[...] = xh
        c_ref[...] = jnp.concatenate(
            [_dot(xh, w1a_ref[...]), _dot(xh, w1b_ref[...])], axis=1)

    return pl.pallas_call(
        body,
        grid=(NPAD // NBLK,),
        in_specs=[_nspec(16), _wspec(16, H), _wspec(1, H), _wspec(H, H),
                  _wspec(1, H), _wspec(1, H), _wspec(1, H), _wspec(H, H),
                  _wspec(H, H)],
        out_specs=[_nspec(H), _nspec(2 * H)],
        out_shape=[jax.ShapeDtypeStruct((NPAD, H), jnp.float32),
                   jax.ShapeDtypeStruct((NPAD, 2 * H), jnp.float32)],
    )(x, w1, b1, w2, b2, g, beta, w1a, w1b)


def _edge_step0(gd, we1, be1, we2, be2, ge, bee, w1c, b1, w2, b2, g, beta):
    def body(g_ref2, we1_ref, be1_ref, we2_ref, be2_ref, ge_ref,
             bee_ref, w1c_ref, b1_ref, w2_ref, b2_ref, g_ref, be_ref,
             msg_ref, enew_ref):
        gg = g_ref2[...]
        d = gg[:, 2 * H:2 * H + 16]
        dist = jnp.sqrt(d[:, 0:1] ** 2 + d[:, 1:2] ** 2)
        dw = jnp.sqrt(d[:, 2:3] ** 2 + d[:, 3:4] ** 2)
        lane = lax.broadcasted_iota(jnp.int32, d.shape, 1)
        feat = jnp.where(lane == 5, dist, d)
        feat = jnp.where(lane == 6, dw, feat)
        feat = jnp.where(lane >= 7, 0.0, feat)
        h = jnp.maximum(_dot(feat, we1_ref[...]) + be1_ref[...], 0.0)
        y = _dot(h, we2_ref[...]) + be2_ref[...]
        eh_v = _ln(y, ge_ref[...], bee_ref[...])
        e1 = _dot(eh_v, w1c_ref[...]) + b1_ref[...]
        h1 = jnp.maximum(gg[:, :H] + e1, 0.0)
        y1 = _dot(h1, w2_ref[...]) + b2_ref[...]
        msg_ref[...] = _ln(y1, g_ref[...], be_ref[...])
        h2 = jnp.maximum(gg[:, H:2 * H] + e1, 0.0)
        y2 = _dot(h2, w2_ref[...]) + b2_ref[...]
        enew_ref[...] = eh_v + _ln(y2, g_ref[...], be_ref[...])

    return pl.pallas_call(
        body,
        grid=(E // EBLK,),
        in_specs=[_espec(3 * H), _wspec(16, H), _wspec(1, H),
                  _wspec(H, H), _wspec(1, H), _wspec(1, H), _wspec(1, H),
                  _wspec(H, H), _wspec(1, H), _wspec(H, H), _wspec(1, H),
                  _wspec(1, H), _wspec(1, H)],
        out_specs=[_espec(H), _espec(H)],
        out_shape=[jax.ShapeDtypeStruct((E, H), jnp.float32),
                   jax.ShapeDtypeStruct((E, H), jnp.float32)],
    )(gd, we1, be1, we2, be2, ge, bee, w1c, b1, w2, b2, g, beta)


def _edge_step(eh, gd, w1c, b1, w2, b2, g, beta):
    def body(eh_ref, gd_ref, w1c_ref, b1_ref, w2_ref, b2_ref,
             g_ref, be_ref, msg_ref, enew_ref):
        eh_v = eh_ref[...]
        e1 = _dot(eh_v, w1c_ref[...]) + b1_ref[...]
        gd = gd_ref[...]
        h1 = jnp.maximum(gd[:, :H] + e1, 0.0)
        y1 = _dot(h1, w2_ref[...]) + b2_ref[...]
        msg_ref[...] = _ln(y1, g_ref[...], be_ref[...])
        h2 = jnp.maximum(gd[:, H:] + e1, 0.0)
        y2 = _dot(h2, w2_ref[...]) + b2_ref[...]
        enew_ref[...] = eh_v + _ln(y2, g_ref[...], be_ref[...])

    return pl.pallas_call(
        body,
        grid=(E // EBLK,),
        in_specs=[_espec(H), _espec(2 * H), _wspec(H, H),
                  _wspec(1, H), _wspec(H, H), _wspec(1, H), _wspec(1, H),
                  _wspec(1, H)],
        out_specs=[_espec(H), _espec(H)],
        out_shape=[jax.ShapeDtypeStruct((E, H), jnp.float32),
                   jax.ShapeDtypeStruct((E, H), jnp.float32)],
    )(eh, gd, w1c, b1, w2, b2, g, beta)


def _node_step(aggr2, xh, w1na, w1nb, b1, w2, b2, g, beta, w1a, w1b):
    def body(a_ref, xh_ref, w1na_ref, w1nb_ref, b1_ref, w2_ref, b2_ref,
             g_ref, be_ref, w1a_ref, w1b_ref, xn_ref, c_ref):
        aggr = a_ref[0] + a_ref[1]
        xh_v = xh_ref[...]
        pre = _dot(aggr, w1na_ref[...]) + _dot(xh_v, w1nb_ref[...]) + b1_ref[...]
        h = jnp.maximum(pre, 0.0)
        y = _dot(h, w2_ref[...]) + b2_ref[...]
        xn = xh_v + _ln(y, g_ref[...], be_ref[...])
        xn_ref[...] = xn
        c_ref[...] = jnp.concatenate(
            [_dot(xn, w1a_ref[...]), _dot(xn, w1b_ref[...])], axis=1)

    aspec = pl.BlockSpec((2, NBLK, H), lambda i: (0, i, 0))
    return pl.pallas_call(
        body,
        grid=(NPAD // NBLK,),
        in_specs=[aspec, _nspec(H), _wspec(H, H), _wspec(H, H), _wspec(1, H),
                  _wspec(H, H), _wspec(1, H), _wspec(1, H), _wspec(1, H),
                  _wspec(H, H), _wspec(H, H)],
        out_specs=[_nspec(H), _nspec(2 * H)],
        out_shape=[jax.ShapeDtypeStruct((NPAD, H), jnp.float32),
                   jax.ShapeDtypeStruct((NPAD, 2 * H), jnp.float32)],
    )(aggr2, xh, w1na, w1nb, b1, w2, b2, g, beta, w1a, w1b)


def _node_step_last(aggr2, xh, w1na, w1nb, b1, w2, b2, g, beta,
                    wd1a, bd1a, wd2a, wd1b, bd1b, wd2b, bdec):
    def body(a_ref, xh_ref, w1na_ref, w1nb_ref, b1_ref, w2_ref, b2_ref,
             g_ref, be_ref, wd1a_ref, bd1a_ref, wd2a_ref, wd1b_ref, bd1b_ref,
             wd2b_ref, bdec_ref, out_ref):
        aggr = a_ref[0] + a_ref[1]
        xh_v = xh_ref[...]
        pre = _dot(aggr, w1na_ref[...]) + _dot(xh_v, w1nb_ref[...]) + b1_ref[...]
        h = jnp.maximum(pre, 0.0)
        y = _dot(h, w2_ref[...]) + b2_ref[...]
        xn = xh_v + _ln(y, g_ref[...], be_ref[...])
        h1 = jnp.maximum(_dot(xn, wd1a_ref[...]) + bd1a_ref[...], 0.0)
        h2 = jnp.maximum(_dot(xn, wd1b_ref[...]) + bd1b_ref[...], 0.0)
        out_ref[...] = _dot(h1, wd2a_ref[...]) + _dot(h2, wd2b_ref[...]) \
            + bdec_ref[...]

    aspec = pl.BlockSpec((2, NBLK, H), lambda i: (0, i, 0))
    return pl.pallas_call(
        body,
        grid=(NPAD // NBLK,),
        in_specs=[aspec, _nspec(H), _wspec(H, H), _wspec(H, H), _wspec(1, H),
                  _wspec(H, H), _wspec(1, H), _wspec(1, H), _wspec(1, H),
                  _wspec(H, H), _wspec(1, H), _wspec(H, H), _wspec(H, H),
                  _wspec(1, H), _wspec(H, H), _wspec(1, H)],
        out_specs=_nspec(H),
        out_shape=jax.ShapeDtypeStruct((NPAD, H), jnp.float32),
    )(aggr2, xh, w1na, w1nb, b1, w2, b2, g, beta,
      wd1a, bd1a, wd2a, wd1b, bd1b, wd2b, bdec)


def _row(v):
    return v.reshape(1, -1)


def _padn(a):
    return jnp.pad(a, ((0, NPAD - N), (0, 0)))


def kernel(world_pos, mesh_pos, prev_world_pos, phi, prev_phi, swelling_phi,
           swelling_phi_rate, swelling_phi_rate_prev, node_type, mat_param,
           edge_index, params):
    f32 = jnp.float32
    src = edge_index[0].astype(jnp.int32).reshape(NW, NCHUNK, KC)
    dst = edge_index[1].astype(jnp.int32).reshape(NW, NCHUNK, KC)

    x = jnp.concatenate(
        [world_pos - prev_world_pos, phi - prev_phi, swelling_phi,
         swelling_phi_rate, swelling_phi_rate_prev, node_type], axis=-1)
    x = _padn(jnp.pad(x, ((0, 0), (0, 6)))).astype(f32)

    p_tab = _padn(jnp.pad(
        jnp.concatenate([mesh_pos, world_pos, phi], axis=-1),
        ((0, 0), (0, H - 5)))).astype(f32)

    ne = params["node_encoder"]
    ee = params["edge_encoder"]
    proc = params["proc"]

    wn1 = jnp.pad(ne["W1"], ((0, 6), (0, 0)))
    we1 = jnp.pad(ee["W1"][jnp.array([0, 1, 3, 4, 6, 2, 5]), :],
                  ((0, 9), (0, 0)))

    ew = [p["edge_mlp"] for p in proc]
    nw_ = [p["node_mlp"] for p in proc]
    w1a = [w["W1"][:H] for w in ew]
    w1b = [w["W1"][H:2 * H] for w in ew]
    w1c = [w["W1"][2 * H:] for w in ew]

    xh, c = _encode_nodes(x, wn1, _row(ne["b1"]), ne["W2"], _row(ne["b2"]),
                          _row(ne["g"]), _row(ne["beta"]), w1a[0], w1b[0])

    zeros = jnp.zeros((NPAD, H), f32)
    wd = params["world_pos_decoder"]
    pdx = params["phi_decoder"]
    wd2a = jnp.pad(wd["W2"], ((0, 0), (0, H - 2)))
    wd2b = jnp.pad(pdx["W2"], ((0, 0), (2, H - 3)))
    bdec = _row(jnp.pad(jnp.concatenate([wd["b2"], pdx["b2"]]), (0, H - 3)))

    for i in range(3):
        e = ew[i]
        nm = nw_[i]
        if i == 0:
            t0 = jnp.concatenate([c, p_tab], axis=1)
            gd = _gather_combine(t0, dst, src, 3 * H)
            msg, eh = _edge_step0(
                gd, we1, _row(ee["b1"]), ee["W2"], _row(ee["b2"]),
                _row(ee["g"]), _row(ee["beta"]), w1c[0], _row(e["b1"]),
                e["W2"], _row(e["b2"]), _row(e["g"]), _row(e["beta"]))
        else:
            gd = _gather_combine(c, dst, src, 2 * H)
            msg, eh = _edge_step(eh, gd, w1c[i], _row(e["b1"]), e["W2"],
                                 _row(e["b2"]), _row(e["g"]), _row(e["beta"]))
        aggr2 = _scatter_add(msg, dst, zeros).reshape(2, NPAD, H)
        if i < 2:
            xh, c = _node_step(aggr2, xh, nm["W1"][:H], nm["W1"][H:],
                               _row(nm["b1"]), nm["W2"], _row(nm["b2"]),
                               _row(nm["g"]), _row(nm["beta"]),
                               w1a[i + 1], w1b[i + 1])
        else:
            out = _node_step_last(aggr2, xh, nm["W1"][:H], nm["W1"][H:],
                                  _row(nm["b1"]), nm["W2"], _row(nm["b2"]),
                                  _row(nm["g"]), _row(nm["beta"]),
                                  wd["W1"], _row(wd["b1"]), wd2a,
                                  pdx["W1"], _row(pdx["b1"]), wd2b, bdec)
    return out[:N, :3]
